# baseline (speedup 1.0000x reference)
import functools

import jax
import jax.numpy as jnp
from jax import lax
from jax.experimental import pallas as pl
from jax.experimental.pallas import tpu as pltpu
from jax.experimental.pallas import tpu_sc as plsc

N = 10000
E = 320000
D = 128

NC = 2
NS = 16
NW = NC * NS
NPAD = 10240
ROWS = NPAD // NS
CHUNK = 80
EPT = E // NW
NIT = EPT // CHUNK
assert NIT % 3 == 2
DEGW = 128

_mesh = plsc.VectorSubcoreMesh(core_axis_name="c", subcore_axis_name="s")


@functools.partial(
    pl.kernel,
    mesh=_mesh,
    out_type=jax.ShapeDtypeStruct((NC, NPAD, DEGW), jnp.float32),
    scratch_types=[
        pltpu.VMEM((NIT, CHUNK), jnp.int32),
        pltpu.VMEM((CHUNK, DEGW), jnp.float32),
        pltpu.VMEM_SHARED((NPAD, DEGW), jnp.float32),
        pltpu.SemaphoreType.DMA,
    ],
)
def _deg_kernel(dst_hbm, ones_hbm, zeros_hbm, out_hbm,
                dst_all, ones_v, acc_sh, sem):
    c = lax.axis_index("c")
    s = lax.axis_index("s")
    wid = c * NS + s
    pltpu.sync_copy(dst_hbm.at[wid], dst_all)
    pltpu.sync_copy(ones_hbm, ones_v)
    pltpu.sync_copy(zeros_hbm, acc_sh.at[pl.ds(s * ROWS, ROWS)])
    plsc.subcore_barrier()

    def body(i, carry):
        pltpu.sync_copy(ones_v, acc_sh.at[dst_all.at[i]], add=True)
        return carry

    lax.fori_loop(0, NIT, body, 0)
    plsc.subcore_barrier()
    pltpu.sync_copy(acc_sh.at[pl.ds(s * ROWS, ROWS)],
                    out_hbm.at[c, pl.ds(s * ROWS, ROWS)])


@functools.partial(
    pl.kernel,
    mesh=_mesh,
    out_type=jax.ShapeDtypeStruct((NC, NPAD, D), jnp.float32),
    scratch_types=[
        pltpu.VMEM((EPT,), jnp.int32),
        pltpu.VMEM((CHUNK,), jnp.int32),
        pltpu.VMEM((CHUNK,), jnp.int32),
        pltpu.VMEM((CHUNK,), jnp.int32),
        pltpu.VMEM((CHUNK, D), jnp.float32),
        pltpu.VMEM((CHUNK, D), jnp.float32),
        pltpu.VMEM((CHUNK, D), jnp.float32),
        pltpu.VMEM_SHARED((NPAD, D), jnp.float32),
        pltpu.SemaphoreType.DMA,
        pltpu.SemaphoreType.DMA,
        pltpu.SemaphoreType.DMA,
        pltpu.SemaphoreType.DMA,
        pltpu.SemaphoreType.DMA,
        pltpu.SemaphoreType.DMA,
        pltpu.SemaphoreType.DMA,
        pltpu.SemaphoreType.DMA,
        pltpu.SemaphoreType.DMA,
    ],
)
def _agg_kernel(h_hbm, src_flat_hbm, dst_hbm, zeros_hbm, out_hbm,
                src_all, dst_a, dst_b, dst_c, rows_a, rows_b, rows_c, acc_sh,
                sem_a, sem_b, sem_c, sem_da, sem_db, sem_dc,
                sem_sa, sem_sb, sem_sc):
    c = lax.axis_index("c")
    s = lax.axis_index("s")
    wid = c * NS + s
    pltpu.sync_copy(src_flat_hbm.at[pl.ds(wid * EPT, EPT)], src_all)
    pltpu.sync_copy(zeros_hbm, acc_sh.at[pl.ds(s * ROWS, ROWS)])
    plsc.subcore_barrier()

    B = ((rows_a, dst_a, sem_a, sem_da, sem_sa),
         (rows_b, dst_b, sem_b, sem_db, sem_sb),
         (rows_c, dst_c, sem_c, sem_dc, sem_sc))

    def gather(i, b):
        rows, dbuf, sg, sd, _ = B[b]
        pltpu.async_copy(h_hbm.at[src_all.at[pl.ds(i * CHUNK, CHUNK)]],
                         rows, sg)
        pltpu.async_copy(dst_hbm.at[wid, i], dbuf, sd)

    def in_wait(b):
        rows, dbuf, sg, sd, _ = B[b]
        pltpu.make_async_copy(h_hbm.at[src_all.at[pl.ds(0, CHUNK)]],
                              rows, sg).wait()
        pltpu.make_async_copy(dst_hbm.at[wid, 0], dbuf, sd).wait()

    def scatter_start(b):
        rows, dbuf, _, _, ss = B[b]
        pltpu.async_copy(rows, acc_sh.at[dbuf], ss, add=True)

    def scatter_wait(b):
        rows, dbuf, _, _, ss = B[b]
        pltpu.make_async_copy(rows, acc_sh.at[dbuf], ss).wait()

    def phase(i, b):
        in_wait(b)
        scatter_start(b)

        bj = (b + 2) % 3

        @pl.when(i + 2 < NIT)
        def _():
            scatter_wait(bj)
            gather(i + 2, bj)

    gather(0, 0)
    gather(1, 1)
    in_wait(0)
    scatter_start(0)
    gather(2, 2)

    def body(k, carry):
        i0 = 3 * k
        phase(i0 + 1, 1)
        phase(i0 + 2, 2)
        phase(i0 + 3, 0)
        return carry

    lax.fori_loop(0, (NIT - 2) // 3, body, 0)
    phase(NIT - 1, (NIT - 1) % 3)
    scatter_wait((NIT - 3) % 3)
    scatter_wait((NIT - 2) % 3)
    scatter_wait((NIT - 1) % 3)
    plsc.subcore_barrier()
    pltpu.sync_copy(acc_sh.at[pl.ds(s * ROWS, ROWS)],
                    out_hbm.at[c, pl.ds(s * ROWS, ROWS)])


def _lin_body(x_ref, w_ref, p_ref, o_ref, dinv_ref):
    deg = p_ref[0, :N, 0:1] + p_ref[1, :N, 0:1] + 1.0
    dinv = lax.rsqrt(deg)
    dinv_ref[...] = dinv
    o_ref[...] = jnp.dot(x_ref[...], w_ref[...],
                         preferred_element_type=jnp.float32) * dinv


def _mid_body(p_ref, h_ref, dinv_ref, b_ref, w_ref, o_ref):
    agg = p_ref[0, :N, :] + p_ref[1, :N, :]
    t = dinv_ref[...] * (agg + h_ref[...]) + b_ref[...]
    t = jnp.maximum(t, 0.0)
    o_ref[...] = jnp.dot(t, w_ref[...],
                         preferred_element_type=jnp.float32) * dinv_ref[...]


def _fin_body(p_ref, h_ref, dinv_ref, b_ref, o_ref):
    agg = p_ref[0, :N, :] + p_ref[1, :N, :]
    o_ref[...] = dinv_ref[...] * (agg + h_ref[...]) + b_ref[...]


_lin_call = pl.pallas_call(
    _lin_body, out_shape=(jax.ShapeDtypeStruct((N, D), jnp.float32),
                          jax.ShapeDtypeStruct((N, 1), jnp.float32)))
_mid_call = pl.pallas_call(
    _mid_body, out_shape=jax.ShapeDtypeStruct((N, D), jnp.float32))
_fin_call = pl.pallas_call(
    _fin_body, out_shape=jax.ShapeDtypeStruct((N, D), jnp.float32))


def kernel(x, edge_index, cache_name, W1, b1, W2, b2):
    src = edge_index[0].astype(jnp.int32)
    dst = edge_index[1].astype(jnp.int32).reshape(NW, NIT, CHUNK)
    ones_deg = jnp.ones((CHUNK, DEGW), jnp.float32)
    zeros_deg = jnp.zeros((ROWS, DEGW), jnp.float32)
    zeros_row = jnp.zeros((ROWS, D), jnp.float32)

    pdeg = _deg_kernel(dst, ones_deg, zeros_deg)
    h1, dinv = _lin_call(x, W1, pdeg)
    agg1 = _agg_kernel(h1, src, dst, zeros_row)
    h2 = _mid_call(agg1, h1, dinv, b1.reshape(1, D), W2)
    agg2 = _agg_kernel(h2, src, dst, zeros_row)
    out = _fin_call(agg2, h2, dinv, b2.reshape(1, D))
    return out


# speedup vs baseline: 1.0242x; 1.0242x over previous
import functools

import jax
import jax.numpy as jnp
from jax import lax
from jax.experimental import pallas as pl
from jax.experimental.pallas import tpu as pltpu
from jax.experimental.pallas import tpu_sc as plsc

N = 10000
E = 320000
D = 128

NC = 2
NS = 16
NW = NC * NS
NPAD = 10240
ROWS = NPAD // NS
CHUNK = 80
EPT = E // NW
NIT = EPT // CHUNK
assert NIT % 3 == 2
DEGW = 128

_mesh = plsc.VectorSubcoreMesh(core_axis_name="c", subcore_axis_name="s")


@functools.partial(
    pl.kernel,
    mesh=_mesh,
    out_type=jax.ShapeDtypeStruct((NC, NPAD, DEGW), jnp.float32),
    scratch_types=[
        pltpu.VMEM((NIT, CHUNK), jnp.int32),
        pltpu.VMEM((CHUNK, DEGW), jnp.float32),
        pltpu.VMEM_SHARED((NPAD, DEGW), jnp.float32),
        pltpu.SemaphoreType.DMA,
    ],
)
def _deg_kernel(dst_hbm, ones_hbm, zeros_hbm, out_hbm,
                dst_all, ones_v, acc_sh, sem):
    c = lax.axis_index("c")
    s = lax.axis_index("s")
    wid = c * NS + s
    pltpu.sync_copy(dst_hbm.at[wid], dst_all)
    pltpu.sync_copy(ones_hbm, ones_v)
    pltpu.sync_copy(zeros_hbm, acc_sh.at[pl.ds(s * ROWS, ROWS)])
    plsc.subcore_barrier()

    def body(i, carry):
        pltpu.sync_copy(ones_v, acc_sh.at[dst_all.at[i]], add=True)
        return carry

    lax.fori_loop(0, NIT, body, 0)
    plsc.subcore_barrier()
    pltpu.sync_copy(acc_sh.at[pl.ds(s * ROWS, ROWS)],
                    out_hbm.at[c, pl.ds(s * ROWS, ROWS)])


@functools.partial(
    pl.kernel,
    mesh=_mesh,
    out_type=jax.ShapeDtypeStruct((NC, NPAD, D), jnp.float32),
    scratch_types=[
        pltpu.VMEM((EPT,), jnp.int32),
        pltpu.VMEM((CHUNK,), jnp.int32),
        pltpu.VMEM((CHUNK,), jnp.int32),
        pltpu.VMEM((CHUNK,), jnp.int32),
        pltpu.VMEM((CHUNK, D), jnp.float32),
        pltpu.VMEM((CHUNK, D), jnp.float32),
        pltpu.VMEM((CHUNK, D), jnp.float32),
        pltpu.VMEM_SHARED((NPAD, D), jnp.float32),
        pltpu.SemaphoreType.DMA,
        pltpu.SemaphoreType.DMA,
        pltpu.SemaphoreType.DMA,
        pltpu.SemaphoreType.DMA,
        pltpu.SemaphoreType.DMA,
        pltpu.SemaphoreType.DMA,
    ],
)
def _agg_kernel(h_hbm, src_flat_hbm, dst_hbm, zeros_hbm, out_hbm,
                src_all, dst_a, dst_b, dst_c, rows_a, rows_b, rows_c, acc_sh,
                sem_a, sem_b, sem_c, sem_da, sem_db, sem_dc):
    c = lax.axis_index("c")
    s = lax.axis_index("s")
    wid = c * NS + s
    pltpu.sync_copy(src_flat_hbm.at[pl.ds(wid * EPT, EPT)], src_all)
    pltpu.sync_copy(zeros_hbm, acc_sh.at[pl.ds(s * ROWS, ROWS)])
    plsc.subcore_barrier()

    B = ((rows_a, dst_a, sem_a, sem_da),
         (rows_b, dst_b, sem_b, sem_db),
         (rows_c, dst_c, sem_c, sem_dc))

    def gather(i, b):
        rows, dbuf, sg, sd = B[b]
        pltpu.async_copy(h_hbm.at[src_all.at[pl.ds(i * CHUNK, CHUNK)]],
                         rows, sg)
        pltpu.async_copy(dst_hbm.at[wid, i], dbuf, sd)

    def phase(i_next, b):
        rows, dbuf, sg, sd = B[b]
        pltpu.make_async_copy(h_hbm.at[src_all.at[pl.ds(0, CHUNK)]],
                              rows, sg).wait()
        pltpu.make_async_copy(dst_hbm.at[wid, 0], dbuf, sd).wait()
        pltpu.sync_copy(rows, acc_sh.at[dbuf], add=True)

        @pl.when(i_next < NIT)
        def _():
            gather(i_next, b)

    gather(0, 0)
    gather(1, 1)
    gather(2, 2)

    def body(k, carry):
        i0 = 3 * k
        phase(i0 + 3, 0)
        phase(i0 + 4, 1)
        phase(i0 + 5, 2)
        return carry

    lax.fori_loop(0, NIT // 3, body, 0)
    phase(NIT, 0)
    phase(NIT, 1)
    plsc.subcore_barrier()
    pltpu.sync_copy(acc_sh.at[pl.ds(s * ROWS, ROWS)],
                    out_hbm.at[c, pl.ds(s * ROWS, ROWS)])


def _lin_body(x_ref, w_ref, p_ref, o_ref, dinv_ref):
    deg = p_ref[0, :N, 0:1] + p_ref[1, :N, 0:1] + 1.0
    dinv = lax.rsqrt(deg)
    dinv_ref[...] = dinv
    o_ref[...] = jnp.dot(x_ref[...], w_ref[...],
                         preferred_element_type=jnp.float32) * dinv


def _mid_body(p_ref, h_ref, dinv_ref, b_ref, w_ref, o_ref):
    agg = p_ref[0, :N, :] + p_ref[1, :N, :]
    t = dinv_ref[...] * (agg + h_ref[...]) + b_ref[...]
    t = jnp.maximum(t, 0.0)
    o_ref[...] = jnp.dot(t, w_ref[...],
                         preferred_element_type=jnp.float32) * dinv_ref[...]


def _fin_body(p_ref, h_ref, dinv_ref, b_ref, o_ref):
    agg = p_ref[0, :N, :] + p_ref[1, :N, :]
    o_ref[...] = dinv_ref[...] * (agg + h_ref[...]) + b_ref[...]


_lin_call = pl.pallas_call(
    _lin_body, out_shape=(jax.ShapeDtypeStruct((N, D), jnp.float32),
                          jax.ShapeDtypeStruct((N, 1), jnp.float32)))
_mid_call = pl.pallas_call(
    _mid_body, out_shape=jax.ShapeDtypeStruct((N, D), jnp.float32))
_fin_call = pl.pallas_call(
    _fin_body, out_shape=jax.ShapeDtypeStruct((N, D), jnp.float32))


def kernel(x, edge_index, cache_name, W1, b1, W2, b2):
    src = edge_index[0].astype(jnp.int32)
    dst = edge_index[1].astype(jnp.int32).reshape(NW, NIT, CHUNK)
    ones_deg = jnp.ones((CHUNK, DEGW), jnp.float32)
    zeros_deg = jnp.zeros((ROWS, DEGW), jnp.float32)
    zeros_row = jnp.zeros((ROWS, D), jnp.float32)

    pdeg = _deg_kernel(dst, ones_deg, zeros_deg)
    h1, dinv = _lin_call(x, W1, pdeg)
    agg1 = _agg_kernel(h1, src, dst, zeros_row)
    h2 = _mid_call(agg1, h1, dinv, b1.reshape(1, D), W2)
    agg2 = _agg_kernel(h2, src, dst, zeros_row)
    out = _fin_call(agg2, h2, dinv, b2.reshape(1, D))
    return out
